# baseline (speedup 1.0000x reference)
import jax
import jax.numpy as jnp
from jax.experimental import pallas as pl
from jax.experimental.pallas import tpu as pltpu

NODE = 256
REL = 4
FOLD = 8
XROWS = NODE * FOLD
XLANES = NODE * REL // FOLD
BLK_J = NODE // FOLD
BLK_R = BLK_J * NODE
NSTEP = FOLD


def _body(xv_ref, w_ref, brow_ref, fc_hbm, fcbias_ref, out_ref, outT3_s,
          buf, sem):
    step = pl.program_id(0)

    def _blk_copy(s_, slot):
        return pltpu.make_async_copy(
            fc_hbm.at[pl.ds(s_ * BLK_R, BLK_R), :], buf.at[slot],
            sem.at[slot])

    @pl.when(step == 0)
    def _prime():
        _blk_copy(0, 0).start()
        _blk_copy(1, 1).start()
        _blk_copy(2, 2).start()

    @pl.when((step > 0) & (step < NSTEP - 2))
    def _next():
        _blk_copy(step + 2, (step + 2) % 3).start()

    @pl.when(step == 0)
    def _gcn():
        mv = (xv_ref[...] != 0.0).astype(jnp.bfloat16)
        li = jax.lax.broadcasted_iota(jnp.int32, (XLANES, BLK_J), 0)
        ji = jax.lax.broadcasted_iota(jnp.int32, (XLANES, BLK_J), 1)
        sel4 = jnp.where((li // REL) == ji, 1.0, 0.0).astype(jnp.bfloat16)
        p = jnp.dot(mv, sel4, preferred_element_type=jnp.float32)
        p3 = p.reshape(NODE, FOLD, BLK_J)
        degf = 1.0 + jnp.sum(p3, axis=0)
        disf = jax.lax.rsqrt(degf)
        dis_col = disf.reshape(NODE, 1)
        dis2048 = jnp.broadcast_to(
            dis_col[:, None, :], (NODE, FOLD, 1)).reshape(XROWS, 1)
        disrowf = jnp.broadcast_to(
            disf[None, :, :], (NODE, FOLD, BLK_J)).reshape(XROWS, BLK_J)
        ri = jax.lax.broadcasted_iota(jnp.int32, (XROWS, BLK_J), 0)
        ci = jax.lax.broadcasted_iota(jnp.int32, (XROWS, BLK_J), 1)
        eyef = jnp.where(ri // FOLD == (ri % FOLD) * BLK_J + ci, 1.0, 0.0)
        gf = (dis2048 * p + eyef * disrowf) * disrowf
        g4 = gf.reshape(NODE, FOLD, BLK_J)
        w_aug = jnp.concatenate([w_ref[...], brow_ref[...]], axis=0)
        ones_row = jnp.full((1, BLK_J), 1.0, jnp.float32)
        for u in range(FOLD):
            g_u = jnp.concatenate([g4[:, u, :], ones_row], axis=0)
            outT3_s[u] = jax.lax.dot_general(
                w_aug, g_u, (((0,), (0,)), ((), ())),
                preferred_element_type=jnp.float32)

    colblk = outT3_s[step]

    _blk_copy(step, step % 3).wait()
    fcb = buf[step % 3]
    partial = jnp.zeros((1, NODE), jnp.float32)
    for jl in range(BLK_J):
        prod = colblk[:, jl:jl + 1] * fcb[jl * NODE:(jl + 1) * NODE, :]
        partial = partial + jnp.sum(prod, axis=0, keepdims=True)

    @pl.when(step == 0)
    def _init():
        out_ref[...] = partial + fcbias_ref[...]

    @pl.when(step > 0)
    def _acc():
        out_ref[...] = out_ref[...] + partial


def kernel(x, W, b_gcn, fc_W, fc_b):
    xv = x.reshape(XROWS, XLANES)
    brow = b_gcn.reshape(1, NODE)
    fcbias = fc_b.reshape(1, NODE)
    return pl.pallas_call(
        _body,
        grid=(NSTEP,),
        in_specs=[
            pl.BlockSpec((XROWS, XLANES), lambda s: (0, 0)),
            pl.BlockSpec((NODE, NODE), lambda s: (0, 0)),
            pl.BlockSpec((1, NODE), lambda s: (0, 0)),
            pl.BlockSpec(memory_space=pltpu.MemorySpace.HBM),
            pl.BlockSpec((1, NODE), lambda s: (0, 0)),
        ],
        out_specs=pl.BlockSpec((1, NODE), lambda s: (0, 0)),
        out_shape=jax.ShapeDtypeStruct((1, NODE), jnp.float32),
        scratch_shapes=[pltpu.VMEM((NSTEP, NODE, BLK_J), jnp.float32),
                        pltpu.VMEM((3, BLK_R, NODE), jnp.float32),
                        pltpu.SemaphoreType.DMA((3,))],
    )(xv, W, brow, fc_W, fcbias)


# speedup vs baseline: 1.0635x; 1.0635x over previous
import jax
import jax.numpy as jnp
from jax.experimental import pallas as pl
from jax.experimental.pallas import tpu as pltpu

NODE = 256
REL = 4
FOLD = 8
XROWS = NODE * FOLD
XLANES = NODE * REL // FOLD
BLK_J = NODE // FOLD
BLK_R = BLK_J * NODE
NSTEP = FOLD


def _body(xv_ref, w_ref, brow_ref, fc_hbm, fcbias_ref, out_ref, outT3_s,
          buf, sem):
    step = pl.program_id(0)

    def _blk_copy(s_, slot):
        return pltpu.make_async_copy(
            fc_hbm.at[pl.ds(s_ * BLK_R, BLK_R), :], buf.at[slot],
            sem.at[slot])

    @pl.when(step == 0)
    def _prime():
        _blk_copy(0, 0).start()
        _blk_copy(1, 1).start()

    @pl.when((step > 0) & (step < NSTEP - 1))
    def _next():
        _blk_copy(step + 1, (step + 1) % 2).start()

    @pl.when(step == 0)
    def _gcn():
        mv = (xv_ref[...] != 0.0).astype(jnp.bfloat16)
        li = jax.lax.broadcasted_iota(jnp.int32, (XLANES, BLK_J), 0)
        ji = jax.lax.broadcasted_iota(jnp.int32, (XLANES, BLK_J), 1)
        sel4 = jnp.where((li // REL) == ji, 1.0, 0.0).astype(jnp.bfloat16)
        p = jnp.dot(mv, sel4, preferred_element_type=jnp.float32)
        p3 = p.reshape(NODE, FOLD, BLK_J)
        degf = 1.0 + jnp.sum(p3, axis=0)
        disf = jax.lax.rsqrt(degf)
        dis_col = disf.reshape(NODE, 1)
        dis2048 = jnp.broadcast_to(
            dis_col[:, None, :], (NODE, FOLD, 1)).reshape(XROWS, 1)
        disrowf = jnp.broadcast_to(
            disf[None, :, :], (NODE, FOLD, BLK_J)).reshape(XROWS, BLK_J)
        ri = jax.lax.broadcasted_iota(jnp.int32, (XROWS, BLK_J), 0)
        ci = jax.lax.broadcasted_iota(jnp.int32, (XROWS, BLK_J), 1)
        eyef = jnp.where(ri // FOLD == (ri % FOLD) * BLK_J + ci, 1.0, 0.0)
        gf = (dis2048 * p + eyef * disrowf) * disrowf
        g4 = gf.reshape(NODE, FOLD, BLK_J)
        w_aug = jnp.concatenate([w_ref[...], brow_ref[...]], axis=0)
        ones_row = jnp.full((1, BLK_J), 1.0, jnp.float32)
        for u in range(FOLD):
            g_u = jnp.concatenate([g4[:, u, :], ones_row], axis=0)
            outT3_s[u] = jax.lax.dot_general(
                w_aug, g_u, (((0,), (0,)), ((), ())),
                preferred_element_type=jnp.float32)

    colblk = outT3_s[step]

    _blk_copy(step, step % 2).wait()
    fcb = buf[step % 2]
    partial = jnp.zeros((1, NODE), jnp.float32)
    for jl in range(BLK_J):
        prod = colblk[:, jl:jl + 1] * fcb[jl * NODE:(jl + 1) * NODE, :]
        partial = partial + jnp.sum(prod, axis=0, keepdims=True)

    @pl.when(step == 0)
    def _init():
        out_ref[...] = partial + fcbias_ref[...]

    @pl.when(step > 0)
    def _acc():
        out_ref[...] = out_ref[...] + partial


def kernel(x, W, b_gcn, fc_W, fc_b):
    xv = x.reshape(XROWS, XLANES)
    brow = b_gcn.reshape(1, NODE)
    fcbias = fc_b.reshape(1, NODE)
    return pl.pallas_call(
        _body,
        grid=(NSTEP,),
        in_specs=[
            pl.BlockSpec((XROWS, XLANES), lambda s: (0, 0)),
            pl.BlockSpec((NODE, NODE), lambda s: (0, 0)),
            pl.BlockSpec((1, NODE), lambda s: (0, 0)),
            pl.BlockSpec(memory_space=pltpu.MemorySpace.HBM),
            pl.BlockSpec((1, NODE), lambda s: (0, 0)),
        ],
        out_specs=pl.BlockSpec((1, NODE), lambda s: (0, 0)),
        out_shape=jax.ShapeDtypeStruct((1, NODE), jnp.float32),
        scratch_shapes=[pltpu.VMEM((NSTEP, NODE, BLK_J), jnp.float32),
                        pltpu.VMEM((2, BLK_R, NODE), jnp.float32),
                        pltpu.SemaphoreType.DMA((2,))],
    )(xv, W, brow, fc_W, fcbias)
